# baseline (speedup 1.0000x reference)
import functools

import numpy as np
import jax
import jax.numpy as jnp
from jax import lax
from jax.experimental import pallas as pl
from jax.experimental.pallas import tpu as pltpu, tpu_sc as plsc

VOCAB = 100000
D_MODEL = 1024
BATCH = 4
SEQ = 4096

_NC = 2
_NS = 16
_NW = _NC * _NS
_POS_PER_W = SEQ // _NW
_C = 8
_K = _POS_PER_W // _C
_LANES = 16
_VECS = D_MODEL // _LANES
_QUARTER = 16


def _pe_table() -> np.ndarray:
    pos = np.arange(SEQ, dtype=np.float32)[:, None]
    two_i = np.arange(0, D_MODEL, 2, dtype=np.float32)
    div = np.power(10000.0, two_i / D_MODEL)
    pe = np.zeros((SEQ, D_MODEL), dtype=np.float32)
    pe[:, 0::2] = np.sin(pos / div)
    pe[:, 1::2] = np.cos(pos / div)
    return pe


_PE = _pe_table()


@functools.partial(
    pl.kernel,
    mesh=plsc.VectorSubcoreMesh(core_axis_name="c", subcore_axis_name="s"),
    out_type=jax.ShapeDtypeStruct((BATCH, SEQ, D_MODEL), jnp.float32),
    scratch_types=(
        [pltpu.VMEM((BATCH, _POS_PER_W), jnp.int32)]
        + [pltpu.VMEM((_C, D_MODEL), jnp.float32)] * 2
        + [pltpu.VMEM((_C, D_MODEL), jnp.float32)] * 8
        + [pltpu.SemaphoreType.DMA] * 4
    ),
)
def _emb_kernel(table_hbm, x_hbm, pe_hbm, out_hbm, idx_all, *scr):
    pe_v = scr[0:2]
    toks = scr[2:10]
    tok = (toks[0:4], toks[4:8])
    gs = scr[10:12]
    ss = scr[12:14]

    wid = lax.axis_index("s") * _NC + lax.axis_index("c")
    pos0 = wid * _POS_PER_W

    for b in range(BATCH):
        pltpu.sync_copy(x_hbm.at[b, pl.ds(pos0, _POS_PER_W)], idx_all.at[b])

    gather_descs = [None, None]
    store_descs = [None, None]

    for k in range(_K + 1):
        if k < _K:
            buf = k % 2
            pos = pos0 + k * _C
            if store_descs[buf] is not None:
                for d in store_descs[buf]:
                    d.wait()
            descs = [pltpu.async_copy(pe_hbm.at[pl.ds(pos, _C)],
                                      pe_v[buf], gs[buf])]
            for b in range(BATCH):
                descs.append(pltpu.async_copy(
                    table_hbm.at[idx_all.at[b, pl.ds(k * _C, _C)]],
                    tok[buf][b], gs[buf]))
            gather_descs[buf] = descs

        if k >= 1:
            cbuf = (k - 1) % 2
            cpos = pos0 + (k - 1) * _C
            for d in gather_descs[cbuf]:
                d.wait()
            pe_b = pe_v[cbuf]
            tk = tok[cbuf]

            def row_body(i, _, pe_b=pe_b, tk=tk):
                def quarter_body(q, _):
                    off = q * (_QUARTER * _LANES)
                    for j in range(_QUARTER):
                        sl = pl.ds(off + j * _LANES, _LANES)
                        p = pe_b[i, sl]
                        for b in range(BATCH):
                            tk[b][i, sl] = tk[b][i, sl] + p
                    return 0

                lax.fori_loop(0, _VECS // _QUARTER, quarter_body, 0)
                return 0

            lax.fori_loop(0, _C, row_body, 0)

            descs = []
            for b in range(BATCH):
                descs.append(pltpu.async_copy(
                    tk[b], out_hbm.at[b, pl.ds(cpos, _C)], ss[cbuf]))
            store_descs[cbuf] = descs

    for buf in range(2):
        if store_descs[buf] is not None:
            for d in store_descs[buf]:
                d.wait()


def kernel(x, token_table):
    x = x.astype(jnp.int32)
    pe = jnp.asarray(_PE)
    return _emb_kernel(token_table, x, pe)
